# Initial kernel scaffold; baseline (speedup 1.0000x reference)
import jax
import jax.numpy as jnp
from jax.experimental import pallas as pl


def kernel(x, weight):
    raise NotImplementedError("write your pallas kernel here")


# speedup vs baseline: 495.0134x; 495.0134x over previous
import functools

import jax
import jax.numpy as jnp
from jax import lax
from jax.experimental import pallas as pl
from jax.experimental.pallas import tpu as pltpu
from jax.experimental.pallas import tpu_sc as plsc

R = 64
RP = 66
T = 16
NC, NS = 2, 16
SLAB = 8
INROWS = SLAB + 2
XSLAB = INROWS * RP * RP
XSLAB_CP = 43584
OUTW = 8 * SLAB * R * T
XP_WORDS = 4 * RP * RP * RP
XP_PAD = 3 * RP * RP * RP + 56 * RP * RP + XSLAB_CP
CHUNK = R * R


def _sc_conv(xp_flat, kp_flat):
    mesh = plsc.VectorSubcoreMesh(core_axis_name="core", subcore_axis_name="sub")

    @functools.partial(
        pl.kernel,
        mesh=mesh,
        compiler_params=pltpu.CompilerParams(needs_layout_passes=False),
        out_type=jax.ShapeDtypeStruct((8 * R * R * R,), jnp.float32),
        scratch_types=[
            pltpu.VMEM((XSLAB_CP,), jnp.float32),
            pltpu.VMEM((OUTW,), jnp.float32),
            pltpu.VMEM((8 * 9 * 6 * 16,), jnp.float32),
        ],
    )
    def k(xp_hbm, kp_hbm, out_hbm, xs, ob, wv):
        wid = lax.axis_index("sub") * NC + lax.axis_index("core")
        cc = wid // 8
        slab = wid % 8
        i0 = slab * SLAB
        pltpu.sync_copy(kp_hbm, wv)
        src = cc * (RP * RP * RP) + i0 * (RP * RP)
        pltpu.sync_copy(xp_hbm.at[pl.ds(src, XSLAB_CP)], xs)
        lanes4 = lax.iota(jnp.int32, 16) * 4

        for p, (oi, oj) in enumerate([(a, b) for a in range(3) for b in range(3)]):
            wvecs = [
                [wv[pl.ds((((o * 3 + oi) * 3 + oj) * 6 + tau) * 16, 16)]
                 for tau in range(6)]
                for o in range(8)
            ]
            first = p == 0

            def body(r, carry, oi=oi, oj=oj, wvecs=wvecs, first=first):
                i = r // R
                y = r - i * R
                base = ((i + oi) * RP + (y + oj)) * RP
                taps = [plsc.load_gather(xs, [lanes4 + (base + tau)])
                        for tau in range(6)]
                for o in range(8):
                    obase = ((o * SLAB + i) * R + y) * T
                    if first:
                        acc = wvecs[o][0] * taps[0]
                        start = 1
                    else:
                        acc = ob[pl.ds(obase, 16)]
                        start = 0
                    for tau in range(start, 6):
                        acc = acc + wvecs[o][tau] * taps[tau]
                    ob[pl.ds(obase, 16)] = acc
                return carry

            lax.fori_loop(0, SLAB * R, body, 0)

        for o in range(8):
            dst = (o * 4 + cc) * (R * R * R // 4) + i0 * (R * R * T // 64)
            pltpu.sync_copy(
                ob.at[pl.ds(o * SLAB * R * T, SLAB * R * T)],
                out_hbm.at[pl.ds(dst, SLAB * R * T)],
            )

    return k(xp_flat, kp_flat)


def kernel(x, weight):
    xpad = jnp.pad(x[0], ((0, 0), (1, 1), (1, 1), (1, 1)))
    kp = jnp.zeros((8, 3, 3, 6), jnp.float32)
    for c2 in range(4):
        kp = kp.at[:, :, :, c2:c2 + 3].add(weight[:, c2])
    xp_flat = jnp.concatenate(
        [xpad.reshape(-1), jnp.zeros((XP_PAD - XP_WORDS + 8,), jnp.float32)]
    )
    out_flat = _sc_conv(xp_flat, jnp.repeat(kp.reshape(-1), 16))
    return out_flat.reshape(1, 8, R, R, R)


# speedup vs baseline: 820.0313x; 1.6566x over previous
import functools

import jax
import jax.numpy as jnp
from jax import lax
from jax.experimental import pallas as pl
from jax.experimental.pallas import tpu as pltpu
from jax.experimental.pallas import tpu_sc as plsc

R = 64
RP = 66
T = 16
NC, NS = 2, 16
SLAB = 8
INROWS = SLAB + 2
XSLAB = INROWS * RP * RP
XSLAB_CP = 43584
OUTW = 8 * SLAB * R * T
XP_WORDS = 4 * RP * RP * RP
XP_PAD = 3 * RP * RP * RP + 56 * RP * RP + XSLAB_CP
CHUNK = R * R


def _sc_conv(xp_flat, kp_flat):
    mesh = plsc.VectorSubcoreMesh(core_axis_name="core", subcore_axis_name="sub")

    @functools.partial(
        pl.kernel,
        mesh=mesh,
        compiler_params=pltpu.CompilerParams(needs_layout_passes=False),
        out_type=jax.ShapeDtypeStruct((8 * R * R * R,), jnp.float32),
        scratch_types=[
            pltpu.VMEM((XSLAB_CP,), jnp.float32),
            pltpu.VMEM((OUTW,), jnp.float32),
            pltpu.VMEM((8 * 9 * 6 * 16,), jnp.float32),
        ],
    )
    def k(xp_hbm, kp_hbm, out_hbm, xs, ob, wv):
        wid = lax.axis_index("sub") * NC + lax.axis_index("core")
        cc = wid // 8
        slab = wid % 8
        i0 = slab * SLAB
        pltpu.sync_copy(kp_hbm, wv)
        src = cc * (RP * RP * RP) + i0 * (RP * RP)
        pltpu.sync_copy(xp_hbm.at[pl.ds(src, XSLAB_CP)], xs)
        lanes4 = lax.iota(jnp.int32, 16) * 4

        for obk in range(2):

            def body(r4, carry, obk=obk):
                i = r4 // 16
                y = (r4 - i * 16) * 4
                acc = [[None] * 4 for _ in range(4)]
                for oi in range(3):
                    for oj in range(3):
                        rowbase = ((i + oi) * RP + (y + oj)) * RP
                        for tau in range(6):
                            w4 = [
                                wv[pl.ds(((((obk * 4 + oidx) * 3 + oi) * 3
                                           + oj) * 6 + tau) * 16, 16)]
                                for oidx in range(4)
                            ]
                            taps = [
                                plsc.load_gather(
                                    xs, [lanes4 + (rowbase + rr * RP + tau)])
                                for rr in range(4)
                            ]
                            for rr in range(4):
                                for oidx in range(4):
                                    prod = w4[oidx] * taps[rr]
                                    if acc[rr][oidx] is None:
                                        acc[rr][oidx] = prod
                                    else:
                                        acc[rr][oidx] = acc[rr][oidx] + prod
                for rr in range(4):
                    for oidx in range(4):
                        o = obk * 4 + oidx
                        obase = ((o * SLAB + i) * R + (y + rr)) * T
                        ob[pl.ds(obase, 16)] = acc[rr][oidx]
                return carry

            lax.fori_loop(0, SLAB * R // 4, body, 0)

        for o in range(8):
            dst = (o * 4 + cc) * (R * R * R // 4) + i0 * (R * R * T // 64)
            pltpu.sync_copy(
                ob.at[pl.ds(o * SLAB * R * T, SLAB * R * T)],
                out_hbm.at[pl.ds(dst, SLAB * R * T)],
            )

    return k(xp_flat, kp_flat)


def kernel(x, weight):
    xpad = jnp.pad(x[0], ((0, 0), (1, 1), (1, 1), (1, 1)))
    kp = jnp.zeros((8, 3, 3, 6), jnp.float32)
    for c2 in range(4):
        kp = kp.at[:, :, :, c2:c2 + 3].add(weight[:, c2])
    xp_flat = jnp.concatenate(
        [xpad.reshape(-1), jnp.zeros((XP_PAD - XP_WORDS + 8,), jnp.float32)]
    )
    out_flat = _sc_conv(xp_flat, jnp.repeat(kp.reshape(-1), 16))
    return out_flat.reshape(1, 8, R, R, R)


# speedup vs baseline: 862.4828x; 1.0518x over previous
import functools

import jax
import jax.numpy as jnp
from jax import lax
from jax.experimental import pallas as pl
from jax.experimental.pallas import tpu as pltpu
from jax.experimental.pallas import tpu_sc as plsc

R = 64
RP = 66
ZP = 72
T = 16
NC, NS = 2, 16
SLAB = 8
INROWS = SLAB + 2
PLANE = RP * ZP
XSLAB = INROWS * PLANE
OUTW = 8 * SLAB * R * T


def _sc_conv(xp_flat, kp_flat):
    mesh = plsc.VectorSubcoreMesh(core_axis_name="core", subcore_axis_name="sub")

    @functools.partial(
        pl.kernel,
        mesh=mesh,
        compiler_params=pltpu.CompilerParams(needs_layout_passes=False),
        out_type=jax.ShapeDtypeStruct((8 * R * R * R,), jnp.float32),
        scratch_types=[
            pltpu.VMEM((XSLAB,), jnp.float32),
            pltpu.VMEM((OUTW,), jnp.float32),
            pltpu.VMEM((8 * 9 * 6 * 16,), jnp.float32),
        ],
    )
    def k(xp_hbm, kp_hbm, out_hbm, xs, ob, wv):
        wid = lax.axis_index("sub") * NC + lax.axis_index("core")
        cc = wid // 8
        slab = wid % 8
        i0 = slab * SLAB
        pltpu.sync_copy(kp_hbm, wv)
        src = cc * (RP * PLANE) + i0 * PLANE
        pltpu.sync_copy(xp_hbm.at[pl.ds(src, XSLAB)], xs)
        lanesT = [lax.iota(jnp.int32, 16) * 4 + tau for tau in range(6)]

        for obk in range(2):

            def body(r4, carry, obk=obk):
                i = r4 // 16
                y = (r4 - i * 16) * 4
                acc = [[None] * 4 for _ in range(4)]
                for oi in range(3):
                    for oj in range(3):
                        rowbase = (i + oi) * PLANE + (y + oj) * ZP
                        for tau in range(6):
                            w4 = [
                                wv[pl.ds(((((obk * 4 + oidx) * 3 + oi) * 3
                                           + oj) * 6 + tau) * 16, 16)]
                                for oidx in range(4)
                            ]
                            taps = [
                                plsc.load_gather(
                                    xs.at[pl.ds(rowbase + rr * ZP, ZP)],
                                    [lanesT[tau]])
                                for rr in range(4)
                            ]
                            for rr in range(4):
                                for oidx in range(4):
                                    prod = w4[oidx] * taps[rr]
                                    if acc[rr][oidx] is None:
                                        acc[rr][oidx] = prod
                                    else:
                                        acc[rr][oidx] = acc[rr][oidx] + prod
                for rr in range(4):
                    for oidx in range(4):
                        o = obk * 4 + oidx
                        obase = ((o * SLAB + i) * R + (y + rr)) * T
                        ob[pl.ds(obase, 16)] = acc[rr][oidx]
                return carry

            lax.fori_loop(0, SLAB * R // 4, body, 0)

        for o in range(8):
            dst = (o * 4 + cc) * (R * R * R // 4) + i0 * (R * R * T // 64)
            pltpu.sync_copy(
                ob.at[pl.ds(o * SLAB * R * T, SLAB * R * T)],
                out_hbm.at[pl.ds(dst, SLAB * R * T)],
            )

    return k(xp_flat, kp_flat)


def kernel(x, weight):
    xpad = jnp.pad(x[0], ((0, 0), (1, 1), (1, 1), (1, 7)))
    kp = jnp.zeros((8, 3, 3, 6), jnp.float32)
    for c2 in range(4):
        kp = kp.at[:, :, :, c2:c2 + 3].add(weight[:, c2])
    out_flat = _sc_conv(xpad.reshape(-1), jnp.repeat(kp.reshape(-1), 16))
    return out_flat.reshape(1, 8, R, R, R)
